# baseline (speedup 1.0000x reference)
import functools

import jax
import jax.numpy as jnp
from jax import lax
from jax.experimental import pallas as pl
from jax.experimental.pallas import tpu as pltpu
from jax.experimental.pallas import tpu_sc as plsc

N_NODES = 10000
NACC = N_NODES + 8
CH = 128
SB = 8
NSUB = 16
NCORE = 2


def _sc_segsum(f2, nch_t):
  nsuper = nch_t // SB
  assert nsuper % 2 == 0
  mesh = plsc.VectorSubcoreMesh(core_axis_name="c", subcore_axis_name="s")

  nzch = 2 * NACC // CH
  zt_off = nzch * CH
  zt = 2 * NACC - zt_off
  nech = N_NODES // CH
  et_off = nech * CH
  et = N_NODES - et_off

  @functools.partial(
      pl.kernel,
      out_type=jax.ShapeDtypeStruct((NCORE, 2 * N_NODES, f2), jnp.float32),
      mesh=mesh,
      compiler_params=pltpu.CompilerParams(use_tc_tiling_on_sc=False),
      scratch_types=[
          pltpu.VMEM((SB * 2, CH), jnp.int32),
          pltpu.VMEM((SB * 2, CH), jnp.int32),
          pltpu.VMEM((CH, f2), jnp.float32),
          pltpu.VMEM((CH, f2), jnp.float32),
          pltpu.VMEM_SHARED((2 * NACC, f2), jnp.float32),
          pltpu.SemaphoreType.DMA,
          pltpu.SemaphoreType.DMA,
          pltpu.SemaphoreType.DMA,
          pltpu.SemaphoreType.DMA,
      ],
  )
  def k(x_hbm, sd_hbm, out_hbm, i0, i1, r0, r1, acc, sg0, sg1, si0, si1):
    c = lax.axis_index("c")
    s = lax.axis_index("s")
    coff = c * N_NODES
    rows = r0

    def zrow(r, carry):
      for j in range(f2 // 16):
        rows[r, pl.ds(j * 16, 16)] = jnp.zeros((16,), jnp.float32)
      return carry
    lax.fori_loop(0, CH, zrow, 0)

    def zcopy(kk, carry):
      @pl.when(kk % NSUB == s)
      def _():
        pltpu.sync_copy(rows, acc.at[pl.ds(kk * CH, CH)])
      return carry
    lax.fori_loop(0, nzch, zcopy, 0)

    @pl.when(nzch % NSUB == s)
    def _():
      pltpu.sync_copy(rows.at[pl.ds(0, zt)], acc.at[pl.ds(zt_off, zt)])
    plsc.subcore_barrier()

    sbase = s * nch_t * 2

    def fire_idx(g, ib, sem):
      pltpu.async_copy(sd_hbm.at[pl.ds(sbase + g * SB * 2, SB * 2)], ib, sem)

    def drain_idx(ib, sem):
      pltpu.make_async_copy(sd_hbm.at[pl.ds(sbase, SB * 2)], ib, sem).wait()

    def adjust(ib, kk):
      for t in range(CH // 16):
        ib[2 * kk, pl.ds(t * 16, 16)] = ib[2 * kk, pl.ds(t * 16, 16)] + coff

    def fire_g(ib, kk, rb, sem):
      pltpu.async_copy(x_hbm.at[ib.at[2 * kk]], rb, sem)

    def drain_g(rb, sem):
      pltpu.make_async_copy(x_hbm.at[i0.at[0]], rb, sem).wait()

    def super_block(icur, inxt, si_nxt, si_cur, g):
      for kk in range(SB):
        rcur, scur = (r0, sg0) if kk % 2 == 0 else (r1, sg1)
        rnxt, snxt = (r1, sg1) if kk % 2 == 0 else (r0, sg0)
        if kk == SB - 1:
          drain_idx(inxt, si_nxt)
          adjust(inxt, 0)
          fire_g(inxt, 0, rnxt, snxt)
        else:
          adjust(icur, kk + 1)
          fire_g(icur, kk + 1, rnxt, snxt)
        drain_g(rcur, scur)
        pltpu.sync_copy(rcur, acc.at[icur.at[2 * kk + 1]], add=True)
      fire_idx(jnp.minimum(g + 2, nsuper - 1), icur, si_cur)

    pltpu.sync_copy(sd_hbm.at[pl.ds(sbase, SB * 2)], i0)
    adjust(i0, 0)
    fire_g(i0, 0, r0, sg0)
    fire_idx(1, i1, si1)

    def body(t, carry):
      super_block(i0, i1, si1, si0, 2 * t)
      super_block(i1, i0, si0, si1, 2 * t + 1)
      return carry
    lax.fori_loop(0, nsuper // 2, body, 0)
    drain_g(r0, sg0)
    drain_idx(i1, si1)
    plsc.subcore_barrier()

    for mp in range(2):
      def ecopy(kk, carry, mp=mp):
        @pl.when(kk % NSUB == s)
        def _():
          pltpu.sync_copy(acc.at[pl.ds(mp * NACC + kk * CH, CH)], rows)
          pltpu.sync_copy(rows,
                          out_hbm.at[c, pl.ds(mp * N_NODES + kk * CH, CH)])
        return carry
      lax.fori_loop(0, nech, ecopy, 0)

      @pl.when((nech + mp) % NSUB == s)
      def _(mp=mp):
        pltpu.sync_copy(acc.at[pl.ds(mp * NACC + et_off, et)],
                        rows.at[pl.ds(0, et)])
        pltpu.sync_copy(rows.at[pl.ds(0, et)],
                        out_hbm.at[c, pl.ds(mp * N_NODES + et_off, et)])

  return k


def _tc_epilogue(a0_ref, a1_ref, w_ref, b_ref, wa_ref, ba_ref, q_ref,
                 out_ref, m0_ref, m1_ref):
  n = m0_ref.shape[0]
  f2 = a0_ref.shape[1]
  wt = w_ref[pl.ds(0, f2)]
  wb = w_ref[pl.ds(f2, f2)]
  b = b_ref[...]

  def gcn(lo):
    acc = jnp.dot(a0_ref[pl.ds(lo, n)], wt,
                  preferred_element_type=jnp.float32)
    acc += jnp.dot(a1_ref[pl.ds(lo, n)], wb,
                   preferred_element_type=jnp.float32)
    return jnp.maximum(acc + b, 0.0)

  m0 = gcn(0)
  m1 = gcn(n)
  m0_ref[...] = m0
  m1_ref[...] = m1
  wa = wa_ref[...]
  ba = ba_ref[...]
  q = q_ref[...]
  h0 = jnp.tanh(jnp.dot(m0, wa, preferred_element_type=jnp.float32) + ba)
  h1 = jnp.tanh(jnp.dot(m1, wa, preferred_element_type=jnp.float32) + ba)
  a0 = jnp.sum(h0 * q) / n
  a1 = jnp.sum(h1 * q) / n
  mx = jnp.maximum(a0, a1)
  e0 = jnp.exp(a0 - mx)
  e1 = jnp.exp(a1 - mx)
  w0 = e0 / (e0 + e1)
  w1 = e1 / (e0 + e1)
  out_ref[...] = w0 * m0 + w1 * m1


def kernel(x, adjs, W, b, Wa, ba, q, sparse):
  del sparse
  p, _, e = adjs.shape
  nfeat = x.shape[1]
  nhid = W.shape[1]
  f2 = nfeat // 2

  x2 = jnp.concatenate([x[:, :f2], x[:, f2:]], axis=0)

  adjs32 = adjs.astype(jnp.int32)
  ept = -(-e // NSUB)
  nch_pm = -(-ept // CH)
  ept_pad = nch_pm * CH
  e_pad = NSUB * ept_pad
  src = jnp.pad(adjs32[:, 0, :], ((0, 0), (0, e_pad - e)))
  dst = jnp.pad(adjs32[:, 1, :], ((0, 0), (0, e_pad - e)),
                constant_values=N_NODES)
  dst = dst + jnp.arange(p, dtype=jnp.int32)[:, None] * NACC
  sd = jnp.stack([src.reshape(p, NSUB, nch_pm, CH),
                  dst.reshape(p, NSUB, nch_pm, CH)], axis=3)
  sd = sd.transpose(1, 0, 2, 3, 4).reshape(NSUB, p * nch_pm, 2, CH)
  nch_t = -(-(p * nch_pm) // (2 * SB)) * (2 * SB)
  padc = jnp.concatenate(
      [jnp.zeros((NSUB, nch_t - p * nch_pm, 1, CH), jnp.int32),
       jnp.full((NSUB, nch_t - p * nch_pm, 1, CH), N_NODES, jnp.int32)],
      axis=2)
  sd = jnp.concatenate([sd, padc], axis=1).reshape(NSUB * nch_t * 2, CH)

  o = _sc_segsum(f2, nch_t)(x2, sd)

  out, m0, m1 = pl.pallas_call(
      _tc_epilogue,
      out_shape=[
          jax.ShapeDtypeStruct((N_NODES, nhid), jnp.float32),
          jax.ShapeDtypeStruct((N_NODES, nhid), jnp.float32),
          jax.ShapeDtypeStruct((N_NODES, nhid), jnp.float32),
      ],
  )(o[0], o[1], W, b.reshape(1, nhid), Wa, ba, q)

  return (out[None], m0, m1)
